# baseline (speedup 1.0000x reference)
import functools

import jax
import jax.numpy as jnp
from jax.experimental import pallas as pl

EPS = 1e-12


def _bn_kernel(v_ref, p_ref, g_ref, b_ref, o_ref):
    x = v_ref[...] + p_ref[...][None, :, :]
    mean = jnp.mean(x, axis=(0, 2), keepdims=True)
    xc = x - mean
    var = jnp.mean(xc * xc, axis=(0, 2), keepdims=True)
    scale = g_ref[...].reshape(1, -1, 1) * jax.lax.rsqrt(var + EPS)
    o_ref[...] = xc * scale + b_ref[...].reshape(1, -1, 1)


@functools.partial(jax.jit, static_argnames=("cb", "interpret"))
def _run(batch_video, row_table, col_table, gamma, beta, cb=16, interpret=False):
    bsz, hsz, height, width = batch_video.shape
    hw = height * width
    v = batch_video.reshape(bsz, hsz, hw)
    r = row_table[:height].reshape(hsz, height)
    c = col_table[:width].reshape(hsz, width)
    p = (r[:, :, None] + c[:, None, :]).reshape(hsz, hw)
    g2 = gamma.reshape(hsz, 1)
    b2 = beta.reshape(hsz, 1)

    out = pl.pallas_call(
        _bn_kernel,
        grid=(hsz // cb,),
        in_specs=[
            pl.BlockSpec((bsz, cb, hw), lambda i: (0, i, 0)),
            pl.BlockSpec((cb, hw), lambda i: (i, 0)),
            pl.BlockSpec((cb, 1), lambda i: (i, 0)),
            pl.BlockSpec((cb, 1), lambda i: (i, 0)),
        ],
        out_specs=pl.BlockSpec((bsz, cb, hw), lambda i: (0, i, 0)),
        out_shape=jax.ShapeDtypeStruct((bsz, hsz, hw), batch_video.dtype),
        interpret=interpret,
    )(v, p, g2, b2)
    return out.reshape(bsz, hsz, height, width)


def kernel(batch_video, row_table, col_table, gamma, beta):
    return _run(batch_video, row_table, col_table, gamma, beta)


# speedup vs baseline: 1.0160x; 1.0160x over previous
import functools

import jax
import jax.numpy as jnp
from jax.experimental import pallas as pl

EPS = 1e-12


def _bn_kernel(v_ref, p_ref, g_ref, b_ref, o_ref):
    x = v_ref[...] + p_ref[...][None, :, :]
    mean = jnp.mean(x, axis=(0, 2), keepdims=True)
    xc = x - mean
    var = jnp.mean(xc * xc, axis=(0, 2), keepdims=True)
    scale = g_ref[...].reshape(1, -1, 1) * jax.lax.rsqrt(var + EPS)
    o_ref[...] = xc * scale + b_ref[...].reshape(1, -1, 1)


@functools.partial(jax.jit, static_argnames=("cb", "interpret"))
def _run(batch_video, row_table, col_table, gamma, beta, cb=16, interpret=False):
    bsz, hsz, height, width = batch_video.shape
    hw = height * width
    v = batch_video.reshape(bsz, hsz, hw)
    r = row_table[:height].reshape(hsz, height)
    c = col_table[:width].reshape(hsz, width)
    p = (r[:, :, None] + c[:, None, :]).reshape(hsz, hw)
    g2 = gamma.reshape(hsz, 1)
    b2 = beta.reshape(hsz, 1)

    out = pl.pallas_call(
        _bn_kernel,
        grid=(hsz // cb,),
        in_specs=[
            pl.BlockSpec((bsz, cb, hw), lambda i: (0, i, 0)),
            pl.BlockSpec((cb, hw), lambda i: (i, 0)),
            pl.BlockSpec((cb, 1), lambda i: (i, 0)),
            pl.BlockSpec((cb, 1), lambda i: (i, 0)),
        ],
        out_specs=pl.BlockSpec((bsz, cb, hw), lambda i: (0, i, 0)),
        out_shape=jax.ShapeDtypeStruct((bsz, hsz, hw), batch_video.dtype),
        interpret=interpret,
    )(v, p, g2, b2)
    return out.reshape(bsz, hsz, height, width)


def kernel(batch_video, row_table, col_table, gamma, beta):
    return _run(batch_video, row_table, col_table, gamma, beta, cb=32)
